# baseline (speedup 1.0000x reference)
import functools

import jax
import jax.numpy as jnp
from jax import lax
from jax.experimental import pallas as pl
from jax.experimental.pallas import tpu as pltpu
from jax.experimental.pallas import tpu_sc as plsc

_B = 2048
_NPART = 5
_N = _B * _NPART
_E = _B * 20
_HID = 512
_SLOPE = 0.2
_EPS = 1e-6

_NC, _NS = 2, 16
_CBLK = 12
_NB = 512


def _make_segsum():
    mesh = plsc.VectorSubcoreMesh(core_axis_name="c", subcore_axis_name="s")
    nch = _E // 128 // _NS
    rows_per_sub = _N // _NS

    @functools.partial(
        pl.kernel,
        out_type=jax.ShapeDtypeStruct((_CBLK * _N, 128), jnp.float32),
        mesh=mesh,
        scratch_types=[
            pltpu.VMEM((nch, 128), jnp.int32),
            pltpu.VMEM((nch, 128), jnp.int32),
            pltpu.VMEM((nch, 128), jnp.int32),
            pltpu.VMEM((128, 128), jnp.float32),
            pltpu.VMEM((128, 128), jnp.float32),
            pltpu.VMEM((40, 128), jnp.float32),
            pltpu.VMEM_SHARED((_N, 128), jnp.float32),
            pltpu.SemaphoreType.DMA,
            pltpu.SemaphoreType.DMA,
        ],
    )
    def segsum(x_hbm, src_hbm, dst_hbm, out_hbm,
               src_v, dst_v, gidx_v, rows0_v, rows1_v,
               zeros_v, acc, sem0, sem1):
        c = lax.axis_index("c")
        s = lax.axis_index("s")
        pltpu.sync_copy(src_hbm.at[s], src_v)
        pltpu.sync_copy(dst_hbm.at[s], dst_v)

        @pl.loop(0, 40)
        def _(r):
            @pl.loop(0, 128, step=16)
            def _(k):
                zeros_v[r, pl.ds(k, 16)] = jnp.zeros((16,), jnp.float32)

        for b in range(_CBLK // _NC):
            bk = b * _NC + c
            off = bk * _N

            @pl.loop(0, rows_per_sub, step=40)
            def _(r0):
                pltpu.sync_copy(zeros_v, acc.at[pl.ds(s * rows_per_sub + r0, 40)])

            @pl.loop(0, nch)
            def _(j):
                @pl.loop(0, 128, step=16)
                def _(k):
                    gidx_v[j, pl.ds(k, 16)] = src_v[j, pl.ds(k, 16)] + off

            plsc.subcore_barrier()

            bufs = (rows0_v, rows1_v)
            sems = (sem0, sem1)
            cps = [pltpu.async_copy(x_hbm.at[gidx_v.at[j]], bufs[j], sems[j])
                   for j in range(2)]
            for j in range(nch):
                bi = j % 2
                cps[bi].wait()
                pltpu.sync_copy(bufs[bi], acc.at[dst_v.at[j]], add=True)
                if j + 2 < nch:
                    cps[bi] = pltpu.async_copy(
                        x_hbm.at[gidx_v.at[j + 2]], bufs[bi], sems[bi])

            plsc.subcore_barrier()
            pltpu.sync_copy(
                acc.at[pl.ds(s * rows_per_sub, rows_per_sub)],
                out_hbm.at[pl.ds(off + s * rows_per_sub, rows_per_sub)])
            plsc.subcore_barrier()

    return segsum


def _make_segsum0():
    mesh = plsc.VectorSubcoreMesh(core_axis_name="c", subcore_axis_name="s")
    nw = _NC * _NS
    nch = _E // 128 // nw
    rows_per_sub = _N // _NS

    @functools.partial(
        pl.kernel,
        out_type=jax.ShapeDtypeStruct((_NC * _N, 128), jnp.float32),
        mesh=mesh,
        scratch_types=[
            pltpu.VMEM((nch, 128), jnp.int32),
            pltpu.VMEM((nch, 128), jnp.int32),
            pltpu.VMEM((128, 128), jnp.float32),
            pltpu.VMEM((40, 128), jnp.float32),
            pltpu.VMEM_SHARED((_N, 128), jnp.float32),
            pltpu.SemaphoreType.DMA,
        ],
    )
    def segsum0(f_hbm, src_hbm, dst_hbm, s0_hbm,
                src_v, dst_v, rows_v, zeros_v, acc, sem):
        c = lax.axis_index("c")
        s = lax.axis_index("s")
        w = s * _NC + c
        pltpu.sync_copy(src_hbm.at[w], src_v)
        pltpu.sync_copy(dst_hbm.at[w], dst_v)

        @pl.loop(0, 40)
        def _(r):
            @pl.loop(0, 128, step=16)
            def _(k):
                zeros_v[r, pl.ds(k, 16)] = jnp.zeros((16,), jnp.float32)

        @pl.loop(0, rows_per_sub, step=40)
        def _(r0):
            pltpu.sync_copy(zeros_v, acc.at[pl.ds(s * rows_per_sub + r0, 40)])

        plsc.subcore_barrier()

        @pl.loop(0, nch)
        def _(j):
            pltpu.async_copy(f_hbm.at[src_v.at[j]], rows_v, sem).wait()
            pltpu.sync_copy(rows_v, acc.at[dst_v.at[j]], add=True)

        plsc.subcore_barrier()
        off = c * _N
        pltpu.sync_copy(acc.at[pl.ds(s * rows_per_sub, rows_per_sub)],
                        s0_hbm.at[pl.ds(off + s * rows_per_sub, rows_per_sub)])

    return segsum0


_SC_CACHE = {}


def _segsum(x_flat, src2d, dst2d):
    if "segsum" not in _SC_CACHE:
        _SC_CACHE["segsum"] = _make_segsum()
    return _SC_CACHE["segsum"](x_flat, src2d, dst2d)


def _segsum0(f16, src2d, dst2d):
    if "segsum0" not in _SC_CACHE:
        _SC_CACHE["segsum0"] = _make_segsum0()
    return _SC_CACHE["segsum0"](f16, src2d, dst2d)


def _feats_body(x0_ref, f_ref):
    blk = x0_ref[...]
    loc = blk[:, :, 0:3]
    vel = blk[:, :, 3:6]
    q = blk[:, :, 6:7]
    cl = loc - jnp.mean(loc, axis=1, keepdims=True)
    a0 = cl[:, :, 1:2] * vel[:, :, 2:3] - cl[:, :, 2:3] * vel[:, :, 1:2]
    a1 = cl[:, :, 2:3] * vel[:, :, 0:1] - cl[:, :, 0:1] * vel[:, :, 2:3]
    a2 = cl[:, :, 0:1] * vel[:, :, 1:2] - cl[:, :, 1:2] * vel[:, :, 0:1]
    ang = jnp.concatenate([a0, a1, a2], axis=2)
    clq = cl * q
    pieces = []
    for a in range(3):
        pieces += [cl[:, :, a:a + 1], vel[:, :, a:a + 1],
                   ang[:, :, a:a + 1], clq[:, :, a:a + 1]]
    pieces.append(jnp.ones_like(blk[:, :, 0:1]))
    pieces.append(jnp.zeros((blk.shape[0], 5, 115), jnp.float32))
    f_ref[...] = jnp.concatenate(pieces, axis=2)


def _feats_call(x0):
    bb = 512
    return pl.pallas_call(
        _feats_body,
        grid=(_B // bb,),
        in_specs=[pl.BlockSpec((bb, 5, 8), lambda i: (i, 0, 0))],
        out_specs=pl.BlockSpec((bb, 5, 128), lambda i: (i, 0, 0)),
        out_shape=jax.ShapeDtypeStruct((_B, 5, 128), jnp.float32),
    )(x0)


def _recip_counts(s0_ref):
    cnt = s0_ref[0, :, 12:13] + s0_ref[1, :, 12:13]
    return 1.0 / jnp.maximum(cnt, 1.0)


def _vn_tail(z_list, d_list):
    dot = sum(z * d for z, d in zip(z_list, d_list))
    dsq = sum(d * d for d in d_list)
    coef = jnp.where(dot >= 0.0, 0.0, dot / (dsq + _EPS))
    return [z - (1.0 - _SLOPE) * coef * d for z, d in zip(z_list, d_list)]


def _layer0_body(f_ref, s_ref, wia_ref, wpa_ref, wdT_ref, b_ref, o_ref):
    bf = jnp.bfloat16
    r = _recip_counts(s_ref)
    f16 = f_ref[:, 0:16]
    s16 = (s_ref[0, :, 0:16] + s_ref[1, :, 0:16]) * r
    z_list, d_list = [], []
    for a in range(3):
        za = (jnp.dot(f16, wia_ref[a], preferred_element_type=jnp.float32)
              + jnp.dot(s16, wpa_ref[a], preferred_element_type=jnp.float32)
              + b_ref[0, :])
        da = jnp.dot(za.astype(bf), wdT_ref[...], preferred_element_type=jnp.float32)
        z_list.append(za)
        d_list.append(da)
    y_list = _vn_tail(z_list, d_list)
    for a in range(3):
        for cb in range(4):
            o_ref[a, cb] = y_list[a][:, cb * 128:(cb + 1) * 128]


def _layer0_call(f128, s0p, wia, wpa, wdT, bias):
    return pl.pallas_call(
        _layer0_body,
        grid=(_N // _NB,),
        in_specs=[
            pl.BlockSpec((_NB, 128), lambda i: (i, 0)),
            pl.BlockSpec((2, _NB, 128), lambda i: (0, i, 0)),
            pl.BlockSpec((3, 16, _HID), lambda i: (0, 0, 0)),
            pl.BlockSpec((3, 16, _HID), lambda i: (0, 0, 0)),
            pl.BlockSpec((_HID, _HID), lambda i: (0, 0)),
            pl.BlockSpec((1, _HID), lambda i: (0, 0)),
        ],
        out_specs=pl.BlockSpec((3, 4, _NB, 128), lambda i: (0, 0, i, 0)),
        out_shape=jax.ShapeDtypeStruct((3, 4, _N, 128), jnp.float32),
    )(f128, s0p, wia, wpa, wdT, bias)


def _id_body(x_ref, wiT_ref, o_ref):
    bf = jnp.bfloat16
    for a in range(3):
        xa = jnp.concatenate([x_ref[a, cb] for cb in range(4)], axis=-1)
        za = jnp.dot(xa.astype(bf), wiT_ref[...],
                     preferred_element_type=jnp.float32)
        for cb in range(4):
            o_ref[a, cb] = za[:, cb * 128:(cb + 1) * 128]


def _id_call(x, wiT):
    return pl.pallas_call(
        _id_body,
        grid=(_N // _NB,),
        in_specs=[
            pl.BlockSpec((3, 4, _NB, 128), lambda i: (0, 0, i, 0)),
            pl.BlockSpec((_HID, _HID), lambda i: (0, 0)),
        ],
        out_specs=pl.BlockSpec((3, 4, _NB, 128), lambda i: (0, 0, i, 0)),
        out_shape=jax.ShapeDtypeStruct((3, 4, _N, 128), jnp.float32),
    )(x, wiT)


def _layer_body(x_ref, zi_ref, s_ref, cnt_ref, wpT_ref, wdT_ref, b_ref, o_ref):
    bf = jnp.bfloat16
    r = _recip_counts(cnt_ref)
    x_list, z_list, d_list = [], [], []
    for a in range(3):
        xa = jnp.concatenate([x_ref[a, cb] for cb in range(4)], axis=-1)
        zia = jnp.concatenate([zi_ref[a, cb] for cb in range(4)], axis=-1)
        sa = jnp.concatenate([s_ref[a, cb] for cb in range(4)], axis=-1)
        za = (zia
              + jnp.dot((sa * r).astype(bf), wpT_ref[...], preferred_element_type=jnp.float32)
              + b_ref[0, :])
        da = jnp.dot(za.astype(bf), wdT_ref[...], preferred_element_type=jnp.float32)
        x_list.append(xa)
        z_list.append(za)
        d_list.append(da)
    y_list = _vn_tail(z_list, d_list)
    for a in range(3):
        ya = y_list[a] + x_list[a]
        for cb in range(4):
            o_ref[a, cb] = ya[:, cb * 128:(cb + 1) * 128]


def _layer_call(x, zi, s, cntp, wpT, wdT, bias):
    return pl.pallas_call(
        _layer_body,
        grid=(_N // _NB,),
        in_specs=[
            pl.BlockSpec((3, 4, _NB, 128), lambda i: (0, 0, i, 0)),
            pl.BlockSpec((3, 4, _NB, 128), lambda i: (0, 0, i, 0)),
            pl.BlockSpec((3, 4, _NB, 128), lambda i: (0, 0, i, 0)),
            pl.BlockSpec((2, _NB, 128), lambda i: (0, i, 0)),
            pl.BlockSpec((_HID, _HID), lambda i: (0, 0)),
            pl.BlockSpec((_HID, _HID), lambda i: (0, 0)),
            pl.BlockSpec((1, _HID), lambda i: (0, 0)),
        ],
        out_specs=pl.BlockSpec((3, 4, _NB, 128), lambda i: (0, 0, i, 0)),
        out_shape=jax.ShapeDtypeStruct((3, 4, _N, 128), jnp.float32),
    )(x, zi, s, cntp, wpT, wdT, bias)


def _final_body(x_ref, s_ref, cnt_ref, wiT_ref, wpT_ref, wdT_ref, b_ref,
                wo_ref, bo_ref, o_ref):
    bf = jnp.bfloat16
    r = _recip_counts(cnt_ref)
    bo = bo_ref[0, 0]
    wo = wo_ref[0, :]
    x_list, z_list, d_list = [], [], []
    for a in range(3):
        xa = jnp.concatenate([x_ref[a, cb] for cb in range(4)], axis=-1)
        sa = jnp.concatenate([s_ref[a, cb] for cb in range(4)], axis=-1)
        za = (jnp.dot(xa.astype(bf), wiT_ref[...], preferred_element_type=jnp.float32)
              + jnp.dot((sa * r).astype(bf), wpT_ref[...], preferred_element_type=jnp.float32)
              + b_ref[0, :])
        da = jnp.dot(za.astype(bf), wdT_ref[...], preferred_element_type=jnp.float32)
        x_list.append(xa)
        z_list.append(za)
        d_list.append(da)
    y_list = _vn_tail(z_list, d_list)
    for a in range(3):
        ya = y_list[a] + x_list[a]
        o_ref[a, :] = jnp.sum(ya * wo, axis=1) + bo


def _final_call(x, s, cntp, wiT, wpT, wdT, bias, wo, bo):
    return pl.pallas_call(
        _final_body,
        grid=(_N // _NB,),
        in_specs=[
            pl.BlockSpec((3, 4, _NB, 128), lambda i: (0, 0, i, 0)),
            pl.BlockSpec((3, 4, _NB, 128), lambda i: (0, 0, i, 0)),
            pl.BlockSpec((2, _NB, 128), lambda i: (0, i, 0)),
            pl.BlockSpec((_HID, _HID), lambda i: (0, 0)),
            pl.BlockSpec((_HID, _HID), lambda i: (0, 0)),
            pl.BlockSpec((_HID, _HID), lambda i: (0, 0)),
            pl.BlockSpec((1, _HID), lambda i: (0, 0)),
            pl.BlockSpec((1, _HID), lambda i: (0, 0)),
            pl.BlockSpec((1, 1), lambda i: (0, 0)),
        ],
        out_specs=pl.BlockSpec((3, _NB), lambda i: (0, i)),
        out_shape=jax.ShapeDtypeStruct((3, _N), jnp.float32),
    )(x, s, cntp, wiT, wpT, wdT, bias, wo, bo)


def _out_body(x_ref, wo_ref, bo_ref, o_ref):
    bo = bo_ref[0, 0]
    wo = wo_ref[0, :]
    for a in range(3):
        xa = jnp.concatenate([x_ref[a, cb] for cb in range(4)], axis=-1)
        o_ref[a, :] = jnp.sum(xa * wo, axis=1) + bo


def _out_call(x, wo, bo):
    return pl.pallas_call(
        _out_body,
        grid=(_N // _NB,),
        in_specs=[
            pl.BlockSpec((3, 4, _NB, 128), lambda i: (0, 0, i, 0)),
            pl.BlockSpec((1, _HID), lambda i: (0, 0)),
            pl.BlockSpec((1, 1), lambda i: (0, 0)),
        ],
        out_specs=pl.BlockSpec((3, _NB), lambda i: (0, i)),
        out_shape=jax.ShapeDtypeStruct((3, _N), jnp.float32),
    )(x, wo, bo)


def kernel(nodes, loc, edges, vel, edge_attr, charges, params):
    del nodes, edge_attr
    f32 = jnp.float32
    src = edges[0].astype(jnp.int32)
    dst = edges[1].astype(jnp.int32)
    nw = _NC * _NS
    src3w = src.reshape(nw, _E // 128 // nw, 128)
    dst3w = dst.reshape(nw, _E // 128 // nw, 128)
    src3s = src.reshape(_NS, _E // 128 // _NS, 128)
    dst3s = dst.reshape(_NS, _E // 128 // _NS, 128)

    x0 = jnp.concatenate(
        [loc, vel, charges, jnp.zeros((_N, 1), f32)], axis=1).reshape(_B, 5, 8)
    f128 = _feats_call(x0).reshape(_N, 128)

    s0_flat = _segsum0(f128, src3w, dst3w)
    s0p = s0_flat.reshape(_NC, _N, 128)

    wi0T = params['Wi0'].T
    wp0T = params['Wp0'].T
    wia = jnp.zeros((3, 16, _HID), f32)
    wpa = jnp.zeros((3, 16, _HID), f32)
    for a in range(3):
        wia = wia.at[a, a * 4:a * 4 + 4, :].set(wi0T)
        wpa = wpa.at[a, a * 4:a * 4 + 4, :].set(wp0T)
    b0 = (params['bi0'] + params['bp0']).reshape(1, _HID)
    bf = jnp.bfloat16
    x = _layer0_call(f128, s0p, wia, wpa, params['Wd0'].T.astype(bf), b0)

    for i in range(1, 3):
        s_flat = _segsum(x.reshape(_CBLK * _N, 128), src3s, dst3s)
        zi = _id_call(x, params['Wi%d' % i].T.astype(bf))
        s = s_flat.reshape(3, 4, _N, 128)
        bias = (params['bi%d' % i] + params['bp%d' % i]).reshape(1, _HID)
        x = _layer_call(x, zi, s, s0p,
                        params['Wp%d' % i].T.astype(bf),
                        params['Wd%d' % i].T.astype(bf), bias)

    s_flat = _segsum(x.reshape(_CBLK * _N, 128), src3s, dst3s)
    s = s_flat.reshape(3, 4, _N, 128)
    bias3 = (params['bi3'] + params['bp3']).reshape(1, _HID)
    o3 = _final_call(x, s, s0p, params['Wi3'].T.astype(bf),
                     params['Wp3'].T.astype(bf), params['Wd3'].T.astype(bf),
                     bias3, params['Wo'], params['bo'].reshape(1, 1))
    return o3.T


# speedup vs baseline: 1.0544x; 1.0544x over previous
import functools

import jax
import jax.numpy as jnp
from jax import lax
from jax.experimental import pallas as pl
from jax.experimental.pallas import tpu as pltpu
from jax.experimental.pallas import tpu_sc as plsc

_B = 2048
_NPART = 5
_N = _B * _NPART
_E = _B * 20
_HID = 512
_SLOPE = 0.2
_EPS = 1e-6

_NC, _NS = 2, 16
_CBLK = 12
_NB = 512


def _make_segsum():
    mesh = plsc.VectorSubcoreMesh(core_axis_name="c", subcore_axis_name="s")
    nch = _E // 128 // _NS
    rows_per_sub = _N // _NS

    @functools.partial(
        pl.kernel,
        out_type=jax.ShapeDtypeStruct((_CBLK * _N, 128), jnp.float32),
        mesh=mesh,
        scratch_types=[
            pltpu.VMEM((nch, 128), jnp.int32),
            pltpu.VMEM((nch, 128), jnp.int32),
            pltpu.VMEM((nch, 128), jnp.int32),
            pltpu.VMEM((128, 128), jnp.float32),
            pltpu.VMEM((128, 128), jnp.float32),
            pltpu.VMEM((40, 128), jnp.float32),
            pltpu.VMEM_SHARED((_N, 128), jnp.float32),
            pltpu.SemaphoreType.DMA,
            pltpu.SemaphoreType.DMA,
        ],
    )
    def segsum(x_hbm, src_hbm, dst_hbm, out_hbm,
               src_v, dst_v, gidx_v, rows0_v, rows1_v,
               zeros_v, acc, sem0, sem1):
        c = lax.axis_index("c")
        s = lax.axis_index("s")
        pltpu.sync_copy(src_hbm.at[s], src_v)
        pltpu.sync_copy(dst_hbm.at[s], dst_v)

        @pl.loop(0, 40)
        def _(r):
            @pl.loop(0, 128, step=16)
            def _(k):
                zeros_v[r, pl.ds(k, 16)] = jnp.zeros((16,), jnp.float32)

        for b in range(_CBLK // _NC):
            bk = b * _NC + c
            off = bk * _N

            @pl.loop(0, rows_per_sub, step=40)
            def _(r0):
                pltpu.sync_copy(zeros_v, acc.at[pl.ds(s * rows_per_sub + r0, 40)])

            @pl.loop(0, nch)
            def _(j):
                @pl.loop(0, 128, step=16)
                def _(k):
                    gidx_v[j, pl.ds(k, 16)] = src_v[j, pl.ds(k, 16)] + off

            plsc.subcore_barrier()

            bufs = (rows0_v, rows1_v)
            sems = (sem0, sem1)
            cps = [pltpu.async_copy(x_hbm.at[gidx_v.at[j]], bufs[j], sems[j])
                   for j in range(2)]
            for j in range(nch):
                bi = j % 2
                cps[bi].wait()
                pltpu.sync_copy(bufs[bi], acc.at[dst_v.at[j]], add=True)
                if j + 2 < nch:
                    cps[bi] = pltpu.async_copy(
                        x_hbm.at[gidx_v.at[j + 2]], bufs[bi], sems[bi])

            plsc.subcore_barrier()
            pltpu.sync_copy(
                acc.at[pl.ds(s * rows_per_sub, rows_per_sub)],
                out_hbm.at[pl.ds(off + s * rows_per_sub, rows_per_sub)])
            plsc.subcore_barrier()

    return segsum


def _make_segsum0():
    mesh = plsc.VectorSubcoreMesh(core_axis_name="c", subcore_axis_name="s")
    nw = _NC * _NS
    nch = _E // 128 // nw
    rows_per_sub = _N // _NS

    @functools.partial(
        pl.kernel,
        out_type=jax.ShapeDtypeStruct((_NC * _N, 128), jnp.float32),
        mesh=mesh,
        scratch_types=[
            pltpu.VMEM((nch, 128), jnp.int32),
            pltpu.VMEM((nch, 128), jnp.int32),
            pltpu.VMEM((128, 128), jnp.float32),
            pltpu.VMEM((40, 128), jnp.float32),
            pltpu.VMEM_SHARED((_N, 128), jnp.float32),
            pltpu.SemaphoreType.DMA,
        ],
    )
    def segsum0(f_hbm, src_hbm, dst_hbm, s0_hbm,
                src_v, dst_v, rows_v, zeros_v, acc, sem):
        c = lax.axis_index("c")
        s = lax.axis_index("s")
        w = s * _NC + c
        pltpu.sync_copy(src_hbm.at[w], src_v)
        pltpu.sync_copy(dst_hbm.at[w], dst_v)

        @pl.loop(0, 40)
        def _(r):
            @pl.loop(0, 128, step=16)
            def _(k):
                zeros_v[r, pl.ds(k, 16)] = jnp.zeros((16,), jnp.float32)

        @pl.loop(0, rows_per_sub, step=40)
        def _(r0):
            pltpu.sync_copy(zeros_v, acc.at[pl.ds(s * rows_per_sub + r0, 40)])

        plsc.subcore_barrier()

        @pl.loop(0, nch)
        def _(j):
            pltpu.async_copy(f_hbm.at[src_v.at[j]], rows_v, sem).wait()
            pltpu.sync_copy(rows_v, acc.at[dst_v.at[j]], add=True)

        plsc.subcore_barrier()
        off = c * _N
        pltpu.sync_copy(acc.at[pl.ds(s * rows_per_sub, rows_per_sub)],
                        s0_hbm.at[pl.ds(off + s * rows_per_sub, rows_per_sub)])

    return segsum0


_SC_CACHE = {}


def _segsum(x_flat, src2d, dst2d):
    if "segsum" not in _SC_CACHE:
        _SC_CACHE["segsum"] = _make_segsum()
    return _SC_CACHE["segsum"](x_flat, src2d, dst2d)


def _segsum0(f16, src2d, dst2d):
    if "segsum0" not in _SC_CACHE:
        _SC_CACHE["segsum0"] = _make_segsum0()
    return _SC_CACHE["segsum0"](f16, src2d, dst2d)


def _feats_body(x0_ref, f_ref):
    blk = x0_ref[...]
    loc = blk[:, :, 0:3]
    vel = blk[:, :, 3:6]
    q = blk[:, :, 6:7]
    cl = loc - jnp.mean(loc, axis=1, keepdims=True)
    a0 = cl[:, :, 1:2] * vel[:, :, 2:3] - cl[:, :, 2:3] * vel[:, :, 1:2]
    a1 = cl[:, :, 2:3] * vel[:, :, 0:1] - cl[:, :, 0:1] * vel[:, :, 2:3]
    a2 = cl[:, :, 0:1] * vel[:, :, 1:2] - cl[:, :, 1:2] * vel[:, :, 0:1]
    ang = jnp.concatenate([a0, a1, a2], axis=2)
    clq = cl * q
    pieces = []
    for a in range(3):
        pieces += [cl[:, :, a:a + 1], vel[:, :, a:a + 1],
                   ang[:, :, a:a + 1], clq[:, :, a:a + 1]]
    pieces.append(jnp.ones_like(blk[:, :, 0:1]))
    pieces.append(jnp.zeros((blk.shape[0], 5, 115), jnp.float32))
    f_ref[...] = jnp.concatenate(pieces, axis=2)


def _feats_call(x0):
    bb = 512
    return pl.pallas_call(
        _feats_body,
        grid=(_B // bb,),
        in_specs=[pl.BlockSpec((bb, 5, 8), lambda i: (i, 0, 0))],
        out_specs=pl.BlockSpec((bb, 5, 128), lambda i: (i, 0, 0)),
        out_shape=jax.ShapeDtypeStruct((_B, 5, 128), jnp.float32),
    )(x0)


def _recip_counts(s0_ref):
    cnt = s0_ref[0, :, 12:13] + s0_ref[1, :, 12:13]
    return 1.0 / jnp.maximum(cnt, 1.0)


def _vn_tail(z_list, d_list):
    dot = sum(z * d for z, d in zip(z_list, d_list))
    dsq = sum(d * d for d in d_list)
    coef = jnp.where(dot >= 0.0, 0.0, dot / (dsq + _EPS))
    return [z - (1.0 - _SLOPE) * coef * d for z, d in zip(z_list, d_list)]


def _layer0_body(f_ref, s_ref, wia_ref, wpa_ref, wdT_ref, b_ref, o_ref):
    bf = jnp.bfloat16
    r = _recip_counts(s_ref)
    f16 = f_ref[:, 0:16]
    s16 = (s_ref[0, :, 0:16] + s_ref[1, :, 0:16]) * r
    z_list, d_list = [], []
    for a in range(3):
        za = (jnp.dot(f16, wia_ref[a], preferred_element_type=jnp.float32)
              + jnp.dot(s16, wpa_ref[a], preferred_element_type=jnp.float32)
              + b_ref[0, :])
        da = jnp.dot(za.astype(bf), wdT_ref[...], preferred_element_type=jnp.float32)
        z_list.append(za)
        d_list.append(da)
    y_list = _vn_tail(z_list, d_list)
    for a in range(3):
        for cb in range(4):
            o_ref[a, cb] = y_list[a][:, cb * 128:(cb + 1) * 128]


def _layer0_call(f128, s0p, wia, wpa, wdT, bias):
    return pl.pallas_call(
        _layer0_body,
        grid=(_N // _NB,),
        in_specs=[
            pl.BlockSpec((_NB, 128), lambda i: (i, 0)),
            pl.BlockSpec((2, _NB, 128), lambda i: (0, i, 0)),
            pl.BlockSpec((3, 16, _HID), lambda i: (0, 0, 0)),
            pl.BlockSpec((3, 16, _HID), lambda i: (0, 0, 0)),
            pl.BlockSpec((_HID, _HID), lambda i: (0, 0)),
            pl.BlockSpec((1, _HID), lambda i: (0, 0)),
        ],
        out_specs=pl.BlockSpec((3, 4, _NB, 128), lambda i: (0, 0, i, 0)),
        out_shape=jax.ShapeDtypeStruct((3, 4, _N, 128), jnp.float32),
    )(f128, s0p, wia, wpa, wdT, bias)


def _layer_body(x_ref, s_ref, cnt_ref, wiT_ref, wpT_ref, wdT_ref, b_ref, o_ref):
    bf = jnp.bfloat16
    r = _recip_counts(cnt_ref)
    x_list, z_list, d_list = [], [], []
    for a in range(3):
        xa = jnp.concatenate([x_ref[a, cb] for cb in range(4)], axis=-1)
        sa = jnp.concatenate([s_ref[a, cb] for cb in range(4)], axis=-1)
        za = (jnp.dot(xa.astype(bf), wiT_ref[...], preferred_element_type=jnp.float32)
              + jnp.dot((sa * r).astype(bf), wpT_ref[...], preferred_element_type=jnp.float32)
              + b_ref[0, :])
        da = jnp.dot(za.astype(bf), wdT_ref[...], preferred_element_type=jnp.float32)
        x_list.append(xa)
        z_list.append(za)
        d_list.append(da)
    y_list = _vn_tail(z_list, d_list)
    for a in range(3):
        ya = y_list[a] + x_list[a]
        for cb in range(4):
            o_ref[a, cb] = ya[:, cb * 128:(cb + 1) * 128]


def _layer_call(x, s, cntp, wiT, wpT, wdT, bias):
    return pl.pallas_call(
        _layer_body,
        grid=(_N // _NB,),
        in_specs=[
            pl.BlockSpec((3, 4, _NB, 128), lambda i: (0, 0, i, 0)),
            pl.BlockSpec((3, 4, _NB, 128), lambda i: (0, 0, i, 0)),
            pl.BlockSpec((2, _NB, 128), lambda i: (0, i, 0)),
            pl.BlockSpec((_HID, _HID), lambda i: (0, 0)),
            pl.BlockSpec((_HID, _HID), lambda i: (0, 0)),
            pl.BlockSpec((_HID, _HID), lambda i: (0, 0)),
            pl.BlockSpec((1, _HID), lambda i: (0, 0)),
        ],
        out_specs=pl.BlockSpec((3, 4, _NB, 128), lambda i: (0, 0, i, 0)),
        out_shape=jax.ShapeDtypeStruct((3, 4, _N, 128), jnp.float32),
    )(x, s, cntp, wiT, wpT, wdT, bias)


def _final_body(x_ref, s_ref, cnt_ref, wiT_ref, wpT_ref, wdT_ref, b_ref,
                wo_ref, bo_ref, o_ref):
    bf = jnp.bfloat16
    r = _recip_counts(cnt_ref)
    bo = bo_ref[0, 0]
    wo = wo_ref[0, :]
    x_list, z_list, d_list = [], [], []
    for a in range(3):
        xa = jnp.concatenate([x_ref[a, cb] for cb in range(4)], axis=-1)
        sa = jnp.concatenate([s_ref[a, cb] for cb in range(4)], axis=-1)
        za = (jnp.dot(xa.astype(bf), wiT_ref[...], preferred_element_type=jnp.float32)
              + jnp.dot((sa * r).astype(bf), wpT_ref[...], preferred_element_type=jnp.float32)
              + b_ref[0, :])
        da = jnp.dot(za.astype(bf), wdT_ref[...], preferred_element_type=jnp.float32)
        x_list.append(xa)
        z_list.append(za)
        d_list.append(da)
    y_list = _vn_tail(z_list, d_list)
    for a in range(3):
        ya = y_list[a] + x_list[a]
        o_ref[a, :] = jnp.sum(ya * wo, axis=1) + bo


def _final_call(x, s, cntp, wiT, wpT, wdT, bias, wo, bo):
    return pl.pallas_call(
        _final_body,
        grid=(_N // _NB,),
        in_specs=[
            pl.BlockSpec((3, 4, _NB, 128), lambda i: (0, 0, i, 0)),
            pl.BlockSpec((3, 4, _NB, 128), lambda i: (0, 0, i, 0)),
            pl.BlockSpec((2, _NB, 128), lambda i: (0, i, 0)),
            pl.BlockSpec((_HID, _HID), lambda i: (0, 0)),
            pl.BlockSpec((_HID, _HID), lambda i: (0, 0)),
            pl.BlockSpec((_HID, _HID), lambda i: (0, 0)),
            pl.BlockSpec((1, _HID), lambda i: (0, 0)),
            pl.BlockSpec((1, _HID), lambda i: (0, 0)),
            pl.BlockSpec((1, 1), lambda i: (0, 0)),
        ],
        out_specs=pl.BlockSpec((3, _NB), lambda i: (0, i)),
        out_shape=jax.ShapeDtypeStruct((3, _N), jnp.float32),
    )(x, s, cntp, wiT, wpT, wdT, bias, wo, bo)


def kernel(nodes, loc, edges, vel, edge_attr, charges, params):
    del nodes, edge_attr
    f32 = jnp.float32
    src = edges[0].astype(jnp.int32)
    dst = edges[1].astype(jnp.int32)
    nw = _NC * _NS
    src3w = src.reshape(nw, _E // 128 // nw, 128)
    dst3w = dst.reshape(nw, _E // 128 // nw, 128)
    src3s = src.reshape(_NS, _E // 128 // _NS, 128)
    dst3s = dst.reshape(_NS, _E // 128 // _NS, 128)

    x0 = jnp.concatenate(
        [loc, vel, charges, jnp.zeros((_N, 1), f32)], axis=1).reshape(_B, 5, 8)
    f128 = _feats_call(x0).reshape(_N, 128)

    s0_flat = _segsum0(f128, src3w, dst3w)
    s0p = s0_flat.reshape(_NC, _N, 128)

    wi0T = params['Wi0'].T
    wp0T = params['Wp0'].T
    wia = jnp.zeros((3, 16, _HID), f32)
    wpa = jnp.zeros((3, 16, _HID), f32)
    for a in range(3):
        wia = wia.at[a, a * 4:a * 4 + 4, :].set(wi0T)
        wpa = wpa.at[a, a * 4:a * 4 + 4, :].set(wp0T)
    b0 = (params['bi0'] + params['bp0']).reshape(1, _HID)
    bf = jnp.bfloat16
    x = _layer0_call(f128, s0p, wia, wpa, params['Wd0'].T.astype(bf), b0)

    for i in range(1, 3):
        s_flat = _segsum(x.reshape(_CBLK * _N, 128), src3s, dst3s)
        s = s_flat.reshape(3, 4, _N, 128)
        bias = (params['bi%d' % i] + params['bp%d' % i]).reshape(1, _HID)
        x = _layer_call(x, s, s0p, params['Wi%d' % i].T.astype(bf),
                        params['Wp%d' % i].T.astype(bf),
                        params['Wd%d' % i].T.astype(bf), bias)

    s_flat = _segsum(x.reshape(_CBLK * _N, 128), src3s, dst3s)
    s = s_flat.reshape(3, 4, _N, 128)
    bias3 = (params['bi3'] + params['bp3']).reshape(1, _HID)
    o3 = _final_call(x, s, s0p, params['Wi3'].T.astype(bf),
                     params['Wp3'].T.astype(bf), params['Wd3'].T.astype(bf),
                     bias3, params['Wo'], params['bo'].reshape(1, 1))
    return o3.T
